# baseline (speedup 1.0000x reference)
import jax
import jax.numpy as jnp
from jax.experimental import pallas as pl
from jax.experimental.pallas import tpu as pltpu

E = 8
K = 2
D = 768
H = 256
H2 = H // 2
N = 16384
EPS = 1e-5

BT = 512


def _normalize(x):
    m = jnp.mean(x, axis=-1, keepdims=True)
    v = jnp.mean((x - m) ** 2, axis=-1, keepdims=True)
    return (x - m) / jnp.sqrt(v + EPS)


def _gelu(x):
    return 0.5 * x * (1.0 + jax.lax.erf(x * (2.0 ** -0.5)))


def _rowstats(h):
    m = jnp.mean(h, axis=-1, keepdims=True)
    v = jnp.mean((h - m) ** 2, axis=-1, keepdims=True)
    return m, jax.lax.rsqrt(v + EPS)


def _moe_block(
    x_ref, g_in_ref, b_in_ref, g_r_ref, b_r_ref, W_r_ref, br_ref,
    ln1gT_ref, ln1b_ref, W1_ref, b1_ref, ln2gT_ref, ln2b_ref,
    W2_ref, b2_ref, ln3gT_ref, ln3bT_ref, W3T_ref, b3_ref,
    out_ref, logits_ref, frac_ref, prob_ref,
    Wr_s, cr_s, W1_s, c1_s, W2_s, c2_s, s2_s, w3_s, s3_s, c3_s,
):
    i = pl.program_id(0)
    nb = pl.num_programs(0)

    @pl.when(i == 0)
    def _():
        Wr_s[...] = g_r_ref[...] * W_r_ref[...]
        cr_s[...] = (jnp.dot(b_r_ref[...], W_r_ref[...],
                             preferred_element_type=jnp.float32)
                     + br_ref[...])
        w3f = ln3gT_ref[...] * W3T_ref[...]
        w3_s[...] = w3f
        s3_s[...] = jnp.sum(w3f, axis=0, keepdims=True)
        c3_s[...] = (jnp.sum(ln3bT_ref[...] * W3T_ref[...],
                             axis=0, keepdims=True)
                     + b3_ref[...])
        for e in range(E):
            W1_s[e] = ln1gT_ref[:, e:e + 1] * W1_ref[e]
            c1_s[e:e + 1, :] = (
                jnp.dot(ln1b_ref[e:e + 1, :], W1_ref[e],
                        preferred_element_type=jnp.float32)
                + b1_ref[e:e + 1, :])
            w2f = ln2gT_ref[:, e:e + 1] * W2_ref[e]
            W2_s[e] = w2f
            s2_s[e:e + 1, :] = jnp.sum(w2f, axis=0, keepdims=True)
            c2_s[e:e + 1, :] = (
                jnp.dot(ln2b_ref[e:e + 1, :], W2_ref[e],
                        preferred_element_type=jnp.float32)
                + b2_ref[e:e + 1, :])

    x = x_ref[...]
    u = _normalize(x) * g_in_ref[...] + b_in_ref[...]
    z = _normalize(u)

    logits = (jnp.dot(z, Wr_s[...], preferred_element_type=jnp.float32)
              + cr_s[...])
    logits_ref[...] = logits

    idx = jax.lax.broadcasted_iota(jnp.int32, (BT, E), 1)
    m1 = jnp.max(logits, axis=1, keepdims=True)
    i1 = jnp.min(jnp.where(logits == m1, idx, E), axis=1, keepdims=True)
    rest = jnp.where(idx == i1, -jnp.inf, logits)
    m2 = jnp.max(rest, axis=1, keepdims=True)
    i2 = jnp.min(jnp.where(rest == m2, idx, E), axis=1, keepdims=True)
    t = jnp.exp(m2 - m1)
    w1 = 1.0 / (1.0 + t)
    w2 = t / (1.0 + t)
    sparse_w = (jnp.where(idx == i1, w1, 0.0)
                + jnp.where(idx == i2, w2, 0.0))
    routed = ((idx == i1) | ((idx == i2) & (w2 > 0.0))).astype(jnp.float32)

    probs = jnp.exp(logits - m1)
    probs = probs / jnp.sum(probs, axis=1, keepdims=True)

    @pl.when(i == 0)
    def _():
        frac_ref[...] = jnp.zeros_like(frac_ref)
        prob_ref[...] = jnp.zeros_like(prob_ref)

    frac_ref[...] += jnp.sum(routed, axis=0, keepdims=True)
    prob_ref[...] += jnp.sum(probs, axis=0, keepdims=True)

    @pl.when(i == nb - 1)
    def _():
        frac_ref[...] *= 1.0 / N
        prob_ref[...] *= 1.0 / N

    dots, mh3s, rh3s = [], [], []
    for e in range(E):
        h = _gelu(jnp.dot(z, W1_s[e], preferred_element_type=jnp.float32)
                  + c1_s[e:e + 1, :])
        mh, rh = _rowstats(h)
        h = (jnp.dot(h, W2_s[e], preferred_element_type=jnp.float32)
             - mh * s2_s[e:e + 1, :]) * rh + c2_s[e:e + 1, :]
        h = _gelu(h)
        mh3, rh3 = _rowstats(h)
        dots.append(jnp.dot(h, w3_s[:, e:e + 1],
                            preferred_element_type=jnp.float32))
        mh3s.append(mh3)
        rh3s.append(rh3)
    dot_c = jnp.concatenate(dots, axis=1)
    mh3_c = jnp.concatenate(mh3s, axis=1)
    rh3_c = jnp.concatenate(rh3s, axis=1)
    ys = rh3_c * (dot_c - mh3_c * s3_s[...]) + c3_s[...]
    out_ref[...] = jnp.sum(ys * sparse_w, axis=1, keepdims=True)


def kernel(x, ln_in_g, ln_in_b, ln_r_g, ln_r_b, W_r, b_r,
           e_ln1_g, e_ln1_b, e_W1, e_b1, e_ln2_g, e_ln2_b,
           e_W2, e_b2, e_ln3_g, e_ln3_b, e_W3, e_b3):
    nb = N // BT

    def rep(shape):
        return pl.BlockSpec(shape, lambda i: (0,) * len(shape))

    out, logits, frac, prob = pl.pallas_call(
        _moe_block,
        grid=(nb,),
        in_specs=[
            pl.BlockSpec((BT, D), lambda i: (i, 0)),
            rep((1, D)), rep((1, D)), rep((D, 1)), rep((1, D)),
            rep((D, E)), rep((1, E)),
            rep((D, E)), rep((E, D)), rep((E, D, H)), rep((E, H)),
            rep((H, E)), rep((E, H)), rep((E, H, H2)), rep((E, H2)),
            rep((H2, E)), rep((H2, E)), rep((H2, E)), rep((1, E)),
        ],
        out_specs=[
            pl.BlockSpec((BT, 1), lambda i: (i, 0)),
            pl.BlockSpec((BT, E), lambda i: (i, 0)),
            pl.BlockSpec((1, E), lambda i: (0, 0)),
            pl.BlockSpec((1, E), lambda i: (0, 0)),
        ],
        out_shape=[
            jax.ShapeDtypeStruct((N, 1), jnp.float32),
            jax.ShapeDtypeStruct((N, E), jnp.float32),
            jax.ShapeDtypeStruct((1, E), jnp.float32),
            jax.ShapeDtypeStruct((1, E), jnp.float32),
        ],
        scratch_shapes=[
            pltpu.VMEM((D, E), jnp.float32),
            pltpu.VMEM((1, E), jnp.float32),
            pltpu.VMEM((E, D, H), jnp.float32),
            pltpu.VMEM((E, H), jnp.float32),
            pltpu.VMEM((E, H, H2), jnp.float32),
            pltpu.VMEM((E, H2), jnp.float32),
            pltpu.VMEM((E, H2), jnp.float32),
            pltpu.VMEM((H2, E), jnp.float32),
            pltpu.VMEM((1, E), jnp.float32),
            pltpu.VMEM((1, E), jnp.float32),
        ],
        compiler_params=pltpu.CompilerParams(
            dimension_semantics=("arbitrary",),
        ),
    )(
        x,
        ln_in_g.reshape(1, D), ln_in_b.reshape(1, D),
        ln_r_g.reshape(D, 1), ln_r_b.reshape(1, D),
        W_r, b_r.reshape(1, E),
        e_ln1_g.T, e_ln1_b, e_W1, e_b1,
        e_ln2_g.T, e_ln2_b, e_W2, e_b2,
        e_ln3_g.T, e_ln3_b.T, e_W3.reshape(E, H2).T, e_b3.reshape(1, E),
    )
    return (out, frac.reshape(E), prob.reshape(E), logits)


# speedup vs baseline: 1.2443x; 1.2443x over previous
import jax
import jax.numpy as jnp
from jax.experimental import pallas as pl
from jax.experimental.pallas import tpu as pltpu

E = 8
K = 2
D = 768
H = 256
H2 = H // 2
N = 16384
EPS = 1e-5

BT = 512


def _normalize(x):
    m = jnp.mean(x, axis=-1, keepdims=True)
    v = jnp.mean((x - m) ** 2, axis=-1, keepdims=True)
    return (x - m) / jnp.sqrt(v + EPS)


def _gelu(x):
    return 0.5 * x * (1.0 + jax.lax.erf(x * (2.0 ** -0.5)))


def _moe_block(
    x_ref, g_in_ref, b_in_ref, g_r_ref, b_r_ref, W_r_ref, br_ref,
    ln1gT_ref, ln1b_ref, W1_ref, b1_ref, ln2g_ref, ln2b_ref,
    W2_ref, b2_ref, ln3g_ref, ln3b_ref, W3_ref, b3_ref,
    out_ref, logits_ref, frac_ref, prob_ref,
    W1_s, c1_s,
):
    i = pl.program_id(0)
    nb = pl.num_programs(0)

    @pl.when(i == 0)
    def _():
        for e in range(E):
            W1_s[e] = ln1gT_ref[:, e:e + 1] * W1_ref[e]
            c1_s[e:e + 1, :] = (
                jnp.dot(ln1b_ref[e:e + 1, :], W1_ref[e],
                        preferred_element_type=jnp.float32)
                + b1_ref[e:e + 1, :])

    x = x_ref[...]
    u = _normalize(x) * g_in_ref[...] + b_in_ref[...]
    z = _normalize(u)

    xr = z * g_r_ref[...] + b_r_ref[...]
    logits = jnp.dot(xr, W_r_ref[...], preferred_element_type=jnp.float32)
    logits = logits + br_ref[...]
    logits_ref[...] = logits

    idx = jax.lax.broadcasted_iota(jnp.int32, (BT, E), 1)
    m1 = jnp.max(logits, axis=1, keepdims=True)
    i1 = jnp.min(jnp.where(logits == m1, idx, E), axis=1, keepdims=True)
    rest = jnp.where(idx == i1, -jnp.inf, logits)
    m2 = jnp.max(rest, axis=1, keepdims=True)
    i2 = jnp.min(jnp.where(rest == m2, idx, E), axis=1, keepdims=True)
    t = jnp.exp(m2 - m1)
    w1 = 1.0 / (1.0 + t)
    w2 = t / (1.0 + t)
    sparse_w = (jnp.where(idx == i1, w1, 0.0)
                + jnp.where(idx == i2, w2, 0.0))
    routed = ((idx == i1) | ((idx == i2) & (w2 > 0.0))).astype(jnp.float32)

    probs = jnp.exp(logits - m1)
    probs = probs / jnp.sum(probs, axis=1, keepdims=True)

    @pl.when(i == 0)
    def _():
        frac_ref[...] = jnp.zeros_like(frac_ref)
        prob_ref[...] = jnp.zeros_like(prob_ref)

    frac_ref[...] += jnp.sum(routed, axis=0, keepdims=True)
    prob_ref[...] += jnp.sum(probs, axis=0, keepdims=True)

    @pl.when(i == nb - 1)
    def _():
        frac_ref[...] *= 1.0 / N
        prob_ref[...] *= 1.0 / N

    ys = []
    for e in range(E):
        h = _gelu(jnp.dot(z, W1_s[e], preferred_element_type=jnp.float32)
                  + c1_s[e:e + 1, :])
        h = _normalize(h) * ln2g_ref[e][None, :] + ln2b_ref[e][None, :]
        h = _gelu(jnp.dot(h, W2_ref[e], preferred_element_type=jnp.float32)
                  + b2_ref[e][None, :])
        h = _normalize(h) * ln3g_ref[e][None, :] + ln3b_ref[e][None, :]
        ys.append(jnp.sum(h * W3_ref[e][None, :], axis=-1, keepdims=True))
    outs = jnp.concatenate(ys, axis=1)
    final = jnp.sum((outs + b3_ref[...]) * sparse_w, axis=1, keepdims=True)
    out_ref[...] = final


def kernel(x, ln_in_g, ln_in_b, ln_r_g, ln_r_b, W_r, b_r,
           e_ln1_g, e_ln1_b, e_W1, e_b1, e_ln2_g, e_ln2_b,
           e_W2, e_b2, e_ln3_g, e_ln3_b, e_W3, e_b3):
    nb = N // BT

    def rep(shape):
        return pl.BlockSpec(shape, lambda i: (0,) * len(shape))

    out, logits, frac, prob = pl.pallas_call(
        _moe_block,
        grid=(nb,),
        in_specs=[
            pl.BlockSpec((BT, D), lambda i: (i, 0)),
            rep((1, D)), rep((1, D)), rep((1, D)), rep((1, D)),
            rep((D, E)), rep((1, E)),
            rep((D, E)), rep((E, D)), rep((E, D, H)), rep((E, H)),
            rep((E, H)), rep((E, H)), rep((E, H, H2)), rep((E, H2)),
            rep((E, H2)), rep((E, H2)), rep((E, H2)), rep((1, E)),
        ],
        out_specs=[
            pl.BlockSpec((BT, 1), lambda i: (i, 0)),
            pl.BlockSpec((BT, E), lambda i: (i, 0)),
            pl.BlockSpec((1, E), lambda i: (0, 0)),
            pl.BlockSpec((1, E), lambda i: (0, 0)),
        ],
        out_shape=[
            jax.ShapeDtypeStruct((N, 1), jnp.float32),
            jax.ShapeDtypeStruct((N, E), jnp.float32),
            jax.ShapeDtypeStruct((1, E), jnp.float32),
            jax.ShapeDtypeStruct((1, E), jnp.float32),
        ],
        scratch_shapes=[
            pltpu.VMEM((E, D, H), jnp.float32),
            pltpu.VMEM((E, H), jnp.float32),
        ],
        compiler_params=pltpu.CompilerParams(
            dimension_semantics=("arbitrary",),
        ),
    )(
        x,
        ln_in_g.reshape(1, D), ln_in_b.reshape(1, D),
        ln_r_g.reshape(1, D), ln_r_b.reshape(1, D),
        W_r, b_r.reshape(1, E),
        e_ln1_g.T, e_ln1_b, e_W1, e_b1,
        e_ln2_g, e_ln2_b, e_W2, e_b2,
        e_ln3_g, e_ln3_b, e_W3.reshape(E, H2), e_b3.reshape(1, E),
    )
    return (out, frac.reshape(E), prob.reshape(E), logits)


# speedup vs baseline: 1.3834x; 1.1118x over previous
import jax
import jax.numpy as jnp
from jax.experimental import pallas as pl
from jax.experimental.pallas import tpu as pltpu

E = 8
K = 2
D = 768
H = 256
H2 = H // 2
N = 16384
EPS = 1e-5

BT = 512


def _normalize(x):
    m = jnp.mean(x, axis=-1, keepdims=True)
    v = jnp.mean(x * x, axis=-1, keepdims=True) - m * m
    return (x - m) * jax.lax.rsqrt(v + EPS)


def _gelu(x):
    return 0.5 * x * (1.0 + jax.lax.erf(x * (2.0 ** -0.5)))


def _moe_block(
    x_ref, g_in_ref, b_in_ref, g_r_ref, b_r_ref, W_r_ref, br_ref,
    ln1gT_ref, ln1b_ref, W1_ref, b1_ref, ln2g_ref, ln2b_ref,
    W2_ref, b2_ref, ln3g_ref, ln3b_ref, W3_ref, b3_ref,
    out_ref, logits_ref, frac_ref, prob_ref,
    W1_s, c1_s, W2_s,
):
    i = pl.program_id(0)
    nb = pl.num_programs(0)

    @pl.when(i == 0)
    def _():
        for e in range(E):
            W1_s[e] = (ln1gT_ref[:, e:e + 1]
                       * W1_ref[e]).astype(jnp.bfloat16)
            c1_s[e:e + 1, :] = (
                jnp.dot(ln1b_ref[e:e + 1, :], W1_ref[e],
                        preferred_element_type=jnp.float32)
                + b1_ref[e:e + 1, :])
            W2_s[e] = W2_ref[e].astype(jnp.bfloat16)

    x = x_ref[...]
    u = _normalize(x) * g_in_ref[...] + b_in_ref[...]
    z = _normalize(u)
    zb = z.astype(jnp.bfloat16)

    xr = z * g_r_ref[...] + b_r_ref[...]
    logits = jnp.dot(xr, W_r_ref[...], preferred_element_type=jnp.float32)
    logits = logits + br_ref[...]
    logits_ref[...] = logits

    idx = jax.lax.broadcasted_iota(jnp.int32, (BT, E), 1)
    m1 = jnp.max(logits, axis=1, keepdims=True)
    i1 = jnp.min(jnp.where(logits == m1, idx, E), axis=1, keepdims=True)
    rest = jnp.where(idx == i1, -jnp.inf, logits)
    m2 = jnp.max(rest, axis=1, keepdims=True)
    i2 = jnp.min(jnp.where(rest == m2, idx, E), axis=1, keepdims=True)
    t = jnp.exp(m2 - m1)
    w1 = 1.0 / (1.0 + t)
    w2 = t / (1.0 + t)
    sparse_w = (jnp.where(idx == i1, w1, 0.0)
                + jnp.where(idx == i2, w2, 0.0))
    routed = ((idx == i1) | ((idx == i2) & (w2 > 0.0))).astype(jnp.float32)

    probs = jnp.exp(logits - m1)
    probs = probs / jnp.sum(probs, axis=1, keepdims=True)

    @pl.when(i == 0)
    def _():
        frac_ref[...] = jnp.zeros_like(frac_ref)
        prob_ref[...] = jnp.zeros_like(prob_ref)

    frac_ref[...] += jnp.sum(routed, axis=0, keepdims=True)
    prob_ref[...] += jnp.sum(probs, axis=0, keepdims=True)

    @pl.when(i == nb - 1)
    def _():
        frac_ref[...] *= 1.0 / N
        prob_ref[...] *= 1.0 / N

    ys = []
    for e in range(E):
        h = _gelu(jnp.dot(zb, W1_s[e], preferred_element_type=jnp.float32)
                  + c1_s[e:e + 1, :])
        h = _normalize(h) * ln2g_ref[e][None, :] + ln2b_ref[e][None, :]
        h = _gelu(jnp.dot(h.astype(jnp.bfloat16), W2_s[e],
                          preferred_element_type=jnp.float32)
                  + b2_ref[e][None, :])
        h = _normalize(h) * ln3g_ref[e][None, :] + ln3b_ref[e][None, :]
        ys.append(jnp.sum(h * W3_ref[e][None, :], axis=-1, keepdims=True))
    outs = jnp.concatenate(ys, axis=1)
    final = jnp.sum((outs + b3_ref[...]) * sparse_w, axis=1, keepdims=True)
    out_ref[...] = final


def kernel(x, ln_in_g, ln_in_b, ln_r_g, ln_r_b, W_r, b_r,
           e_ln1_g, e_ln1_b, e_W1, e_b1, e_ln2_g, e_ln2_b,
           e_W2, e_b2, e_ln3_g, e_ln3_b, e_W3, e_b3):
    nb = N // BT

    def rep(shape):
        return pl.BlockSpec(shape, lambda i: (0,) * len(shape))

    out, logits, frac, prob = pl.pallas_call(
        _moe_block,
        grid=(nb,),
        in_specs=[
            pl.BlockSpec((BT, D), lambda i: (i, 0)),
            rep((1, D)), rep((1, D)), rep((1, D)), rep((1, D)),
            rep((D, E)), rep((1, E)),
            rep((D, E)), rep((E, D)), rep((E, D, H)), rep((E, H)),
            rep((E, H)), rep((E, H)), rep((E, H, H2)), rep((E, H2)),
            rep((E, H2)), rep((E, H2)), rep((E, H2)), rep((1, E)),
        ],
        out_specs=[
            pl.BlockSpec((BT, 1), lambda i: (i, 0)),
            pl.BlockSpec((BT, E), lambda i: (i, 0)),
            pl.BlockSpec((1, E), lambda i: (0, 0)),
            pl.BlockSpec((1, E), lambda i: (0, 0)),
        ],
        out_shape=[
            jax.ShapeDtypeStruct((N, 1), jnp.float32),
            jax.ShapeDtypeStruct((N, E), jnp.float32),
            jax.ShapeDtypeStruct((1, E), jnp.float32),
            jax.ShapeDtypeStruct((1, E), jnp.float32),
        ],
        scratch_shapes=[
            pltpu.VMEM((E, D, H), jnp.bfloat16),
            pltpu.VMEM((E, H), jnp.float32),
            pltpu.VMEM((E, H, H2), jnp.bfloat16),
        ],
        compiler_params=pltpu.CompilerParams(
            dimension_semantics=("arbitrary",),
        ),
    )(
        x,
        ln_in_g.reshape(1, D), ln_in_b.reshape(1, D),
        ln_r_g.reshape(1, D), ln_r_b.reshape(1, D),
        W_r, b_r.reshape(1, E),
        e_ln1_g.T, e_ln1_b, e_W1, e_b1,
        e_ln2_g, e_ln2_b, e_W2, e_b2,
        e_ln3_g, e_ln3_b, e_W3.reshape(E, H2), e_b3.reshape(1, E),
    )
    return (out, frac.reshape(E), prob.reshape(E), logits)


# speedup vs baseline: 1.5294x; 1.1055x over previous
import jax
import jax.numpy as jnp
from jax.experimental import pallas as pl
from jax.experimental.pallas import tpu as pltpu

E = 8
K = 2
D = 768
H = 256
H2 = H // 2
N = 16384
EPS = 1e-5

BT = 1024


def _normalize(x):
    m = jnp.mean(x, axis=-1, keepdims=True)
    v = jnp.mean(x * x, axis=-1, keepdims=True) - m * m
    return (x - m) * jax.lax.rsqrt(v + EPS)


def _gelu(x):
    return 0.5 * x * (1.0 + jax.lax.erf(x * (2.0 ** -0.5)))


def _moe_block(
    x_ref, g_in_ref, b_in_ref, g_r_ref, b_r_ref, W_r_ref, br_ref,
    ln1gT_ref, ln1b_ref, W1_ref, b1_ref, ln2g_ref, ln2b_ref,
    W2_ref, b2_ref, ln3g_ref, ln3b_ref, W3_ref, b3_ref,
    out_ref, logits_ref, frac_ref, prob_ref,
    W1_s, c1_s, W2_s,
):
    i = pl.program_id(0)
    nb = pl.num_programs(0)

    @pl.when(i == 0)
    def _():
        for e in range(E):
            W1_s[e] = (ln1gT_ref[:, e:e + 1]
                       * W1_ref[e]).astype(jnp.bfloat16)
            c1_s[e:e + 1, :] = (
                jnp.dot(ln1b_ref[e:e + 1, :], W1_ref[e],
                        preferred_element_type=jnp.float32)
                + b1_ref[e:e + 1, :])
            W2_s[e] = W2_ref[e].astype(jnp.bfloat16)

    x = x_ref[...]
    u = _normalize(x) * g_in_ref[...] + b_in_ref[...]
    z = _normalize(u)
    zb = z.astype(jnp.bfloat16)

    xr = z * g_r_ref[...] + b_r_ref[...]
    logits = jnp.dot(xr, W_r_ref[...], preferred_element_type=jnp.float32)
    logits = logits + br_ref[...]
    logits_ref[...] = logits

    idx = jax.lax.broadcasted_iota(jnp.int32, (BT, E), 1)
    m1 = jnp.max(logits, axis=1, keepdims=True)
    i1 = jnp.min(jnp.where(logits == m1, idx, E), axis=1, keepdims=True)
    rest = jnp.where(idx == i1, -jnp.inf, logits)
    m2 = jnp.max(rest, axis=1, keepdims=True)
    i2 = jnp.min(jnp.where(rest == m2, idx, E), axis=1, keepdims=True)
    t = jnp.exp(m2 - m1)
    w1 = 1.0 / (1.0 + t)
    w2 = t / (1.0 + t)
    sparse_w = (jnp.where(idx == i1, w1, 0.0)
                + jnp.where(idx == i2, w2, 0.0))
    routed = ((idx == i1) | ((idx == i2) & (w2 > 0.0))).astype(jnp.float32)

    probs = jnp.exp(logits - m1)
    probs = probs / jnp.sum(probs, axis=1, keepdims=True)

    @pl.when(i == 0)
    def _():
        frac_ref[...] = jnp.zeros_like(frac_ref)
        prob_ref[...] = jnp.zeros_like(prob_ref)

    frac_ref[...] += jnp.sum(routed, axis=0, keepdims=True)
    prob_ref[...] += jnp.sum(probs, axis=0, keepdims=True)

    @pl.when(i == nb - 1)
    def _():
        frac_ref[...] *= 1.0 / N
        prob_ref[...] *= 1.0 / N

    ys = []
    for e in range(E):
        h = _gelu(jnp.dot(zb, W1_s[e], preferred_element_type=jnp.float32)
                  + c1_s[e:e + 1, :])
        h = _normalize(h) * ln2g_ref[e][None, :] + ln2b_ref[e][None, :]
        h = _gelu(jnp.dot(h.astype(jnp.bfloat16), W2_s[e],
                          preferred_element_type=jnp.float32)
                  + b2_ref[e][None, :])
        h = _normalize(h) * ln3g_ref[e][None, :] + ln3b_ref[e][None, :]
        ys.append(jnp.sum(h * W3_ref[e][None, :], axis=-1, keepdims=True))
    outs = jnp.concatenate(ys, axis=1)
    final = jnp.sum((outs + b3_ref[...]) * sparse_w, axis=1, keepdims=True)
    out_ref[...] = final


def kernel(x, ln_in_g, ln_in_b, ln_r_g, ln_r_b, W_r, b_r,
           e_ln1_g, e_ln1_b, e_W1, e_b1, e_ln2_g, e_ln2_b,
           e_W2, e_b2, e_ln3_g, e_ln3_b, e_W3, e_b3):
    nb = N // BT

    def rep(shape):
        return pl.BlockSpec(shape, lambda i: (0,) * len(shape))

    out, logits, frac, prob = pl.pallas_call(
        _moe_block,
        grid=(nb,),
        in_specs=[
            pl.BlockSpec((BT, D), lambda i: (i, 0)),
            rep((1, D)), rep((1, D)), rep((1, D)), rep((1, D)),
            rep((D, E)), rep((1, E)),
            rep((D, E)), rep((E, D)), rep((E, D, H)), rep((E, H)),
            rep((E, H)), rep((E, H)), rep((E, H, H2)), rep((E, H2)),
            rep((E, H2)), rep((E, H2)), rep((E, H2)), rep((1, E)),
        ],
        out_specs=[
            pl.BlockSpec((BT, 1), lambda i: (i, 0)),
            pl.BlockSpec((BT, E), lambda i: (i, 0)),
            pl.BlockSpec((1, E), lambda i: (0, 0)),
            pl.BlockSpec((1, E), lambda i: (0, 0)),
        ],
        out_shape=[
            jax.ShapeDtypeStruct((N, 1), jnp.float32),
            jax.ShapeDtypeStruct((N, E), jnp.float32),
            jax.ShapeDtypeStruct((1, E), jnp.float32),
            jax.ShapeDtypeStruct((1, E), jnp.float32),
        ],
        scratch_shapes=[
            pltpu.VMEM((E, D, H), jnp.bfloat16),
            pltpu.VMEM((E, H), jnp.float32),
            pltpu.VMEM((E, H, H2), jnp.bfloat16),
        ],
        compiler_params=pltpu.CompilerParams(
            dimension_semantics=("arbitrary",),
        ),
    )(
        x,
        ln_in_g.reshape(1, D), ln_in_b.reshape(1, D),
        ln_r_g.reshape(1, D), ln_r_b.reshape(1, D),
        W_r, b_r.reshape(1, E),
        e_ln1_g.T, e_ln1_b, e_W1, e_b1,
        e_ln2_g, e_ln2_b, e_W2, e_b2,
        e_ln3_g, e_ln3_b, e_W3.reshape(E, H2), e_b3.reshape(1, E),
    )
    return (out, frac.reshape(E), prob.reshape(E), logits)


# speedup vs baseline: 1.5360x; 1.0043x over previous
import jax
import jax.numpy as jnp
from jax.experimental import pallas as pl
from jax.experimental.pallas import tpu as pltpu

E = 8
K = 2
D = 768
H = 256
H2 = H // 2
N = 16384
EPS = 1e-5

BT = 2048


def _normalize(x):
    m = jnp.mean(x, axis=-1, keepdims=True)
    v = jnp.mean(x * x, axis=-1, keepdims=True) - m * m
    return (x - m) * jax.lax.rsqrt(v + EPS)


def _gelu(x):
    return 0.5 * x * (1.0 + jax.lax.erf(x * (2.0 ** -0.5)))


def _moe_block(
    x_ref, g_in_ref, b_in_ref, g_r_ref, b_r_ref, W_r_ref, br_ref,
    ln1gT_ref, ln1b_ref, W1_ref, b1_ref, ln2g_ref, ln2b_ref,
    W2_ref, b2_ref, ln3g_ref, ln3b_ref, W3_ref, b3_ref,
    out_ref, logits_ref, frac_ref, prob_ref,
    W1_s, c1_s, W2_s,
):
    i = pl.program_id(0)
    nb = pl.num_programs(0)

    @pl.when(i == 0)
    def _():
        for e in range(E):
            W1_s[e] = (ln1gT_ref[:, e:e + 1]
                       * W1_ref[e]).astype(jnp.bfloat16)
            c1_s[e:e + 1, :] = (
                jnp.dot(ln1b_ref[e:e + 1, :], W1_ref[e],
                        preferred_element_type=jnp.float32)
                + b1_ref[e:e + 1, :])
            W2_s[e] = W2_ref[e].astype(jnp.bfloat16)

    x = x_ref[...]
    u = _normalize(x) * g_in_ref[...] + b_in_ref[...]
    z = _normalize(u)
    zb = z.astype(jnp.bfloat16)

    xr = z * g_r_ref[...] + b_r_ref[...]
    logits = jnp.dot(xr, W_r_ref[...], preferred_element_type=jnp.float32)
    logits = logits + br_ref[...]
    logits_ref[...] = logits

    idx = jax.lax.broadcasted_iota(jnp.int32, (BT, E), 1)
    m1 = jnp.max(logits, axis=1, keepdims=True)
    i1 = jnp.min(jnp.where(logits == m1, idx, E), axis=1, keepdims=True)
    rest = jnp.where(idx == i1, -jnp.inf, logits)
    m2 = jnp.max(rest, axis=1, keepdims=True)
    i2 = jnp.min(jnp.where(rest == m2, idx, E), axis=1, keepdims=True)
    t = jnp.exp(m2 - m1)
    w1 = 1.0 / (1.0 + t)
    w2 = t / (1.0 + t)
    sparse_w = (jnp.where(idx == i1, w1, 0.0)
                + jnp.where(idx == i2, w2, 0.0))
    routed = ((idx == i1) | ((idx == i2) & (w2 > 0.0))).astype(jnp.float32)

    probs = jnp.exp(logits - m1)
    probs = probs / jnp.sum(probs, axis=1, keepdims=True)

    @pl.when(i == 0)
    def _():
        frac_ref[...] = jnp.zeros_like(frac_ref)
        prob_ref[...] = jnp.zeros_like(prob_ref)

    frac_ref[...] += jnp.sum(routed, axis=0, keepdims=True)
    prob_ref[...] += jnp.sum(probs, axis=0, keepdims=True)

    @pl.when(i == nb - 1)
    def _():
        frac_ref[...] *= 1.0 / N
        prob_ref[...] *= 1.0 / N

    ys = []
    for e in range(E):
        h = _gelu(jnp.dot(zb, W1_s[e], preferred_element_type=jnp.float32)
                  + c1_s[e:e + 1, :])
        h = _normalize(h) * ln2g_ref[e][None, :] + ln2b_ref[e][None, :]
        h = _gelu(jnp.dot(h.astype(jnp.bfloat16), W2_s[e],
                          preferred_element_type=jnp.float32)
                  + b2_ref[e][None, :])
        h = _normalize(h) * ln3g_ref[e][None, :] + ln3b_ref[e][None, :]
        ys.append(jnp.sum(h * W3_ref[e][None, :], axis=-1, keepdims=True))
    outs = jnp.concatenate(ys, axis=1)
    final = jnp.sum((outs + b3_ref[...]) * sparse_w, axis=1, keepdims=True)
    out_ref[...] = final


def kernel(x, ln_in_g, ln_in_b, ln_r_g, ln_r_b, W_r, b_r,
           e_ln1_g, e_ln1_b, e_W1, e_b1, e_ln2_g, e_ln2_b,
           e_W2, e_b2, e_ln3_g, e_ln3_b, e_W3, e_b3):
    nb = N // BT

    def rep(shape):
        return pl.BlockSpec(shape, lambda i: (0,) * len(shape))

    out, logits, frac, prob = pl.pallas_call(
        _moe_block,
        grid=(nb,),
        in_specs=[
            pl.BlockSpec((BT, D), lambda i: (i, 0)),
            rep((1, D)), rep((1, D)), rep((1, D)), rep((1, D)),
            rep((D, E)), rep((1, E)),
            rep((D, E)), rep((E, D)), rep((E, D, H)), rep((E, H)),
            rep((E, H)), rep((E, H)), rep((E, H, H2)), rep((E, H2)),
            rep((E, H2)), rep((E, H2)), rep((E, H2)), rep((1, E)),
        ],
        out_specs=[
            pl.BlockSpec((BT, 1), lambda i: (i, 0)),
            pl.BlockSpec((BT, E), lambda i: (i, 0)),
            pl.BlockSpec((1, E), lambda i: (0, 0)),
            pl.BlockSpec((1, E), lambda i: (0, 0)),
        ],
        out_shape=[
            jax.ShapeDtypeStruct((N, 1), jnp.float32),
            jax.ShapeDtypeStruct((N, E), jnp.float32),
            jax.ShapeDtypeStruct((1, E), jnp.float32),
            jax.ShapeDtypeStruct((1, E), jnp.float32),
        ],
        scratch_shapes=[
            pltpu.VMEM((E, D, H), jnp.bfloat16),
            pltpu.VMEM((E, H), jnp.float32),
            pltpu.VMEM((E, H, H2), jnp.bfloat16),
        ],
        compiler_params=pltpu.CompilerParams(
            dimension_semantics=("arbitrary",),
        ),
    )(
        x,
        ln_in_g.reshape(1, D), ln_in_b.reshape(1, D),
        ln_r_g.reshape(1, D), ln_r_b.reshape(1, D),
        W_r, b_r.reshape(1, E),
        e_ln1_g.T, e_ln1_b, e_W1, e_b1,
        e_ln2_g, e_ln2_b, e_W2, e_b2,
        e_ln3_g, e_ln3_b, e_W3.reshape(E, H2), e_b3.reshape(1, E),
    )
    return (out, frac.reshape(E), prob.reshape(E), logits)


# speedup vs baseline: 1.6451x; 1.0711x over previous
import jax
import jax.numpy as jnp
from jax.experimental import pallas as pl
from jax.experimental.pallas import tpu as pltpu

E = 8
K = 2
D = 768
H = 256
H2 = H // 2
N = 16384
EPS = 1e-5

BT = 2048


def _normalize(x):
    m = jnp.mean(x, axis=-1, keepdims=True)
    v = jnp.mean(x * x, axis=-1, keepdims=True) - m * m
    return (x - m) * jax.lax.rsqrt(v + EPS)


def _gelu(x):
    return 0.5 * x * (1.0 + jax.lax.erf(x * (2.0 ** -0.5)))


def _rowstats(h):
    m = jnp.mean(h, axis=-1, keepdims=True)
    v = jnp.mean(h * h, axis=-1, keepdims=True) - m * m
    return m, jax.lax.rsqrt(v + EPS)


def _moe_block(
    x_ref, g_in_ref, b_in_ref, g_r_ref, b_r_ref, W_r_ref, br_ref,
    ln1gT_ref, ln1b_ref, W1_ref, b1_ref, ln2gT_ref, ln2b_ref,
    W2_ref, b2_ref, ln3g_ref, ln3gT_ref, ln3bT_ref, W3_ref, W3T_ref,
    b3_ref,
    out_ref, logits_ref, frac_ref, prob_ref,
    W1_s, c1_s, W2_s, s2_s, c2_s, w3_s, s3_s, c3_s,
):
    i = pl.program_id(0)
    nb = pl.num_programs(0)

    @pl.when(i == 0)
    def _():
        s3_s[...] = jnp.sum(ln3gT_ref[...] * W3T_ref[...],
                            axis=0, keepdims=True)
        c3_s[...] = (jnp.sum(ln3bT_ref[...] * W3T_ref[...],
                             axis=0, keepdims=True)
                     + b3_ref[...])
        for e in range(E):
            W1_s[e] = (ln1gT_ref[:, e:e + 1]
                       * W1_ref[e]).astype(jnp.bfloat16)
            c1_s[e:e + 1, :] = (
                jnp.dot(ln1b_ref[e:e + 1, :], W1_ref[e],
                        preferred_element_type=jnp.float32)
                + b1_ref[e:e + 1, :])
            w2f = ln2gT_ref[:, e:e + 1] * W2_ref[e]
            W2_s[e] = w2f.astype(jnp.bfloat16)
            s2_s[e:e + 1, :] = jnp.sum(w2f, axis=0, keepdims=True)
            c2_s[e:e + 1, :] = (
                jnp.dot(ln2b_ref[e:e + 1, :], W2_ref[e],
                        preferred_element_type=jnp.float32)
                + b2_ref[e:e + 1, :])
            w3_s[e:e + 1, :] = ln3g_ref[e:e + 1, :] * W3_ref[e:e + 1, :]

    x = x_ref[...]
    u = _normalize(x) * g_in_ref[...] + b_in_ref[...]
    z = _normalize(u)
    zb = z.astype(jnp.bfloat16)

    xr = z * g_r_ref[...] + b_r_ref[...]
    logits = jnp.dot(xr, W_r_ref[...], preferred_element_type=jnp.float32)
    logits = logits + br_ref[...]
    logits_ref[...] = logits

    idx = jax.lax.broadcasted_iota(jnp.int32, (BT, E), 1)
    m1 = jnp.max(logits, axis=1, keepdims=True)
    i1 = jnp.min(jnp.where(logits == m1, idx, E), axis=1, keepdims=True)
    rest = jnp.where(idx == i1, -jnp.inf, logits)
    m2 = jnp.max(rest, axis=1, keepdims=True)
    i2 = jnp.min(jnp.where(rest == m2, idx, E), axis=1, keepdims=True)
    t = jnp.exp(m2 - m1)
    w1 = 1.0 / (1.0 + t)
    w2 = t / (1.0 + t)
    sparse_w = (jnp.where(idx == i1, w1, 0.0)
                + jnp.where(idx == i2, w2, 0.0))
    routed = ((idx == i1) | ((idx == i2) & (w2 > 0.0))).astype(jnp.float32)

    probs = jnp.exp(logits - m1)
    probs = probs / jnp.sum(probs, axis=1, keepdims=True)

    @pl.when(i == 0)
    def _():
        frac_ref[...] = jnp.zeros_like(frac_ref)
        prob_ref[...] = jnp.zeros_like(prob_ref)

    frac_ref[...] += jnp.sum(routed, axis=0, keepdims=True)
    prob_ref[...] += jnp.sum(probs, axis=0, keepdims=True)

    @pl.when(i == nb - 1)
    def _():
        frac_ref[...] *= 1.0 / N
        prob_ref[...] *= 1.0 / N

    dots, mh3s, rh3s = [], [], []
    for e in range(E):
        h = _gelu(jnp.dot(zb, W1_s[e], preferred_element_type=jnp.float32)
                  + c1_s[e:e + 1, :])
        mh, rh = _rowstats(h)
        h = (jnp.dot(h.astype(jnp.bfloat16), W2_s[e],
                     preferred_element_type=jnp.float32)
             - mh * s2_s[e:e + 1, :]) * rh + c2_s[e:e + 1, :]
        h = _gelu(h)
        mh3, rh3 = _rowstats(h)
        dots.append(jnp.sum(h * w3_s[e:e + 1, :], axis=-1, keepdims=True))
        mh3s.append(mh3)
        rh3s.append(rh3)
    dot_c = jnp.concatenate(dots, axis=1)
    mh3_c = jnp.concatenate(mh3s, axis=1)
    rh3_c = jnp.concatenate(rh3s, axis=1)
    ys = rh3_c * (dot_c - mh3_c * s3_s[...]) + c3_s[...]
    out_ref[...] = jnp.sum(ys * sparse_w, axis=1, keepdims=True)


def kernel(x, ln_in_g, ln_in_b, ln_r_g, ln_r_b, W_r, b_r,
           e_ln1_g, e_ln1_b, e_W1, e_b1, e_ln2_g, e_ln2_b,
           e_W2, e_b2, e_ln3_g, e_ln3_b, e_W3, e_b3):
    nb = N // BT

    def rep(shape):
        return pl.BlockSpec(shape, lambda i: (0,) * len(shape))

    out, logits, frac, prob = pl.pallas_call(
        _moe_block,
        grid=(nb,),
        in_specs=[
            pl.BlockSpec((BT, D), lambda i: (i, 0)),
            rep((1, D)), rep((1, D)), rep((1, D)), rep((1, D)),
            rep((D, E)), rep((1, E)),
            rep((D, E)), rep((E, D)), rep((E, D, H)), rep((E, H)),
            rep((H, E)), rep((E, H)), rep((E, H, H2)), rep((E, H2)),
            rep((E, H2)), rep((H2, E)), rep((H2, E)), rep((E, H2)),
            rep((H2, E)), rep((1, E)),
        ],
        out_specs=[
            pl.BlockSpec((BT, 1), lambda i: (i, 0)),
            pl.BlockSpec((BT, E), lambda i: (i, 0)),
            pl.BlockSpec((1, E), lambda i: (0, 0)),
            pl.BlockSpec((1, E), lambda i: (0, 0)),
        ],
        out_shape=[
            jax.ShapeDtypeStruct((N, 1), jnp.float32),
            jax.ShapeDtypeStruct((N, E), jnp.float32),
            jax.ShapeDtypeStruct((1, E), jnp.float32),
            jax.ShapeDtypeStruct((1, E), jnp.float32),
        ],
        scratch_shapes=[
            pltpu.VMEM((E, D, H), jnp.bfloat16),
            pltpu.VMEM((E, H), jnp.float32),
            pltpu.VMEM((E, H, H2), jnp.bfloat16),
            pltpu.VMEM((E, H2), jnp.float32),
            pltpu.VMEM((E, H2), jnp.float32),
            pltpu.VMEM((E, H2), jnp.float32),
            pltpu.VMEM((1, E), jnp.float32),
            pltpu.VMEM((1, E), jnp.float32),
        ],
        compiler_params=pltpu.CompilerParams(
            dimension_semantics=("arbitrary",),
        ),
    )(
        x,
        ln_in_g.reshape(1, D), ln_in_b.reshape(1, D),
        ln_r_g.reshape(1, D), ln_r_b.reshape(1, D),
        W_r, b_r.reshape(1, E),
        e_ln1_g.T, e_ln1_b, e_W1, e_b1,
        e_ln2_g.T, e_ln2_b, e_W2, e_b2,
        e_ln3_g, e_ln3_g.T, e_ln3_b.T,
        e_W3.reshape(E, H2), e_W3.reshape(E, H2).T, e_b3.reshape(1, E),
    )
    return (out, frac.reshape(E), prob.reshape(E), logits)
